# baseline (speedup 1.0000x reference)
import functools

import jax
import jax.numpy as jnp
from jax import lax
from jax.experimental import pallas as pl
from jax.experimental.pallas import tpu as pltpu
from jax.experimental.pallas import tpu_sc as plsc

W = 4096
H = 4096
LANES = 16
NUM_WORKERS = 32
CHUNK = 128
NBUF = 8


def _make_kernel(n_samples):
    per_tile = n_samples // NUM_WORKERS
    n_chunks = per_tile // CHUNK
    assert n_chunks % NBUF == 0
    n_groups = n_chunks // NBUF
    mesh = plsc.VectorSubcoreMesh(core_axis_name="c", subcore_axis_name="s")

    @functools.partial(
        pl.kernel,
        mesh=mesh,
        out_type=jax.ShapeDtypeStruct((n_samples,), jnp.float32),
        scratch_types=[
            pltpu.VMEM((2 * per_tile,), jnp.float32),
            pltpu.VMEM((per_tile,), jnp.float32),
            pltpu.VMEM((NBUF, 4, CHUNK), jnp.int32),
            pltpu.VMEM((NBUF, 4, CHUNK), jnp.float32),
            pltpu.VMEM((NBUF, 4, CHUNK), jnp.float32),
        ] + [pltpu.SemaphoreType.DMA] * NBUF,
    )
    def tex_kernel(tex_hbm, xy_hbm, out_hbm,
                   xy_v, out_v, idx_v, val_v, wt_v, *sems):
        wid = lax.axis_index("s") * 2 + lax.axis_index("c")
        base = wid * per_tile
        pltpu.sync_copy(xy_hbm.at[pl.ds(2 * base, 2 * per_tile)], xy_v)
        iota = lax.iota(jnp.int32, LANES)
        perm_e = (iota * 2) % LANES
        perm_o = (iota * 2 + 1) % LANES
        lo_half = iota < (LANES // 2)

        def compute_and_fire(ci, slot):
            off = ci * CHUNK
            for i in range(CHUNK // LANES):
                s2 = 2 * (off + i * LANES)
                d = pl.ds(i * LANES, LANES)
                a = xy_v[pl.ds(s2, LANES)]
                b = xy_v[pl.ds(s2 + LANES, LANES)]
                xf = jnp.where(lo_half,
                               a.at[perm_e].get(mode="promise_in_bounds"),
                               b.at[perm_e].get(mode="promise_in_bounds"))
                yf = jnp.where(lo_half,
                               a.at[perm_o].get(mode="promise_in_bounds"),
                               b.at[perm_o].get(mode="promise_in_bounds"))
                gx = ((xf * 2.0 - 1.0) + 1.0) * 0.5 * (W - 1)
                gy = ((yf * 2.0 - 1.0) + 1.0) * 0.5 * (H - 1)
                x0 = gx.astype(jnp.int32)
                y0 = gy.astype(jnp.int32)
                wx1 = gx - x0.astype(jnp.float32)
                wy1 = gy - y0.astype(jnp.float32)
                wx0 = 1.0 - wx1
                wy0 = 1.0 - wy1
                flat = y0 * W + x0
                idx_v[slot, 0, d] = flat
                idx_v[slot, 1, d] = flat + 1
                idx_v[slot, 2, d] = flat + W
                idx_v[slot, 3, d] = flat + (W + 1)
                wt_v[slot, 0, d] = wy0 * wx0
                wt_v[slot, 1, d] = wy0 * wx1
                wt_v[slot, 2, d] = wy1 * wx0
                wt_v[slot, 3, d] = wy1 * wx1
            for c in range(4):
                pltpu.async_copy(tex_hbm.at[idx_v.at[slot, c]],
                                 val_v.at[slot, c], sems[slot])

        def drain_and_combine(ci, slot):
            for c in range(4):
                pltpu.make_async_copy(tex_hbm.at[idx_v.at[slot, c]],
                                      val_v.at[slot, c], sems[slot]).wait()
            off = ci * CHUNK
            for i in range(CHUNK // LANES):
                d = pl.ds(i * LANES, LANES)
                out_v[pl.ds(off + i * LANES, LANES)] = (
                    val_v[slot, 0, d] * wt_v[slot, 0, d]
                    + val_v[slot, 1, d] * wt_v[slot, 1, d]
                    + val_v[slot, 2, d] * wt_v[slot, 2, d]
                    + val_v[slot, 3, d] * wt_v[slot, 3, d]
                )

        for b in range(NBUF - 1):
            compute_and_fire(b, b)

        def loop_body(j, carry):
            cb = j * NBUF
            for b in range(NBUF):
                compute_and_fire(cb + b + (NBUF - 1), (b + NBUF - 1) % NBUF)
                drain_and_combine(cb + b, b)
            return carry

        lax.fori_loop(0, n_groups - 1, loop_body, 0)
        cb = (n_groups - 1) * NBUF
        compute_and_fire(n_chunks - 1, (NBUF - 1) % NBUF)
        for b in range(NBUF):
            drain_and_combine(cb + b, b)

        pltpu.sync_copy(out_v, out_hbm.at[pl.ds(base, per_tile)])

    return tex_kernel


def kernel(x, layer1):
    n, ho, wo = x.shape[0], x.shape[1], x.shape[2]
    n_samples = n * ho * wo
    xy = x.reshape(2 * n_samples)
    tex = layer1.reshape(W * H)
    out = _make_kernel(n_samples)(tex, xy)
    return out.reshape(n, 1, ho, wo)


# speedup vs baseline: 6.4462x; 6.4462x over previous
import functools

import jax
import jax.numpy as jnp
from jax import lax
from jax.experimental import pallas as pl
from jax.experimental.pallas import tpu as pltpu
from jax.experimental.pallas import tpu_sc as plsc

W = 4096
H = 4096
LANES = 16
NUM_WORKERS = 32
CHUNK = 128
NBUF = 8


def _make_kernel(n_samples):
    per_tile = n_samples // NUM_WORKERS
    n_chunks = per_tile // CHUNK
    assert n_chunks % NBUF == 0
    n_groups = n_chunks // NBUF
    mesh = plsc.VectorSubcoreMesh(core_axis_name="c", subcore_axis_name="s")

    @functools.partial(
        pl.kernel,
        mesh=mesh,
        out_type=jax.ShapeDtypeStruct((n_samples,), jnp.float32),
        scratch_types=[
            pltpu.VMEM((per_tile,), jnp.float32),
            pltpu.VMEM((per_tile,), jnp.float32),
            pltpu.VMEM((per_tile,), jnp.float32),
            pltpu.VMEM((NBUF, 4, CHUNK), jnp.int32),
            pltpu.VMEM((NBUF, 4, CHUNK), jnp.float32),
            pltpu.VMEM((NBUF, 4, CHUNK), jnp.float32),
        ] + [pltpu.SemaphoreType.DMA] * NBUF,
    )
    def tex_kernel(tex_hbm, xs_hbm, ys_hbm, out_hbm,
                   xs_v, ys_v, out_v, idx_v, val_v, wt_v, *sems):
        wid = lax.axis_index("s") * 2 + lax.axis_index("c")
        base = wid * per_tile
        pltpu.sync_copy(xs_hbm.at[pl.ds(base, per_tile)], xs_v)
        pltpu.sync_copy(ys_hbm.at[pl.ds(base, per_tile)], ys_v)

        def compute_and_fire(ci, slot):
            off = ci * CHUNK
            for i in range(CHUNK // LANES):
                s = off + i * LANES
                d = pl.ds(i * LANES, LANES)
                xf = xs_v[pl.ds(s, LANES)]
                yf = ys_v[pl.ds(s, LANES)]
                gx = ((xf * 2.0 - 1.0) + 1.0) * 0.5 * (W - 1)
                gy = ((yf * 2.0 - 1.0) + 1.0) * 0.5 * (H - 1)
                x0 = gx.astype(jnp.int32)
                y0 = gy.astype(jnp.int32)
                wx1 = gx - x0.astype(jnp.float32)
                wy1 = gy - y0.astype(jnp.float32)
                wx0 = 1.0 - wx1
                wy0 = 1.0 - wy1
                flat = y0 * W + x0
                idx_v[slot, 0, d] = flat
                idx_v[slot, 1, d] = flat + 1
                idx_v[slot, 2, d] = flat + W
                idx_v[slot, 3, d] = flat + (W + 1)
                wt_v[slot, 0, d] = wy0 * wx0
                wt_v[slot, 1, d] = wy0 * wx1
                wt_v[slot, 2, d] = wy1 * wx0
                wt_v[slot, 3, d] = wy1 * wx1
            for c in range(4):
                pltpu.async_copy(tex_hbm.at[idx_v.at[slot, c]],
                                 val_v.at[slot, c], sems[slot])

        def drain_and_combine(ci, slot):
            for c in range(4):
                pltpu.make_async_copy(tex_hbm.at[idx_v.at[slot, c]],
                                      val_v.at[slot, c], sems[slot]).wait()
            off = ci * CHUNK
            for i in range(CHUNK // LANES):
                d = pl.ds(i * LANES, LANES)
                out_v[pl.ds(off + i * LANES, LANES)] = (
                    val_v[slot, 0, d] * wt_v[slot, 0, d]
                    + val_v[slot, 1, d] * wt_v[slot, 1, d]
                    + val_v[slot, 2, d] * wt_v[slot, 2, d]
                    + val_v[slot, 3, d] * wt_v[slot, 3, d]
                )

        for b in range(NBUF - 1):
            compute_and_fire(b, b)

        def loop_body(j, carry):
            cb = j * NBUF
            for b in range(NBUF):
                compute_and_fire(cb + b + (NBUF - 1), (b + NBUF - 1) % NBUF)
                drain_and_combine(cb + b, b)
            return carry

        lax.fori_loop(0, n_groups - 1, loop_body, 0)
        cb = (n_groups - 1) * NBUF
        compute_and_fire(n_chunks - 1, (NBUF - 1) % NBUF)
        for b in range(NBUF):
            drain_and_combine(cb + b, b)

        pltpu.sync_copy(out_v, out_hbm.at[pl.ds(base, per_tile)])

    return tex_kernel


def kernel(x, layer1):
    n, ho, wo = x.shape[0], x.shape[1], x.shape[2]
    n_samples = n * ho * wo
    xs = x[..., 0].reshape(n_samples)
    ys = x[..., 1].reshape(n_samples)
    tex = layer1.reshape(W * H)
    out = _make_kernel(n_samples)(tex, xs, ys)
    return out.reshape(n, 1, ho, wo)


# speedup vs baseline: 6.5824x; 1.0211x over previous
import functools

import jax
import jax.numpy as jnp
from jax import lax
from jax.experimental import pallas as pl
from jax.experimental.pallas import tpu as pltpu
from jax.experimental.pallas import tpu_sc as plsc

W = 4096
H = 4096
LANES = 16
NUM_WORKERS = 32
CHUNK = 128
NBUF = 4


def _make_kernel(n_samples):
    per_tile = n_samples // NUM_WORKERS
    n_chunks = per_tile // CHUNK
    assert n_chunks % NBUF == 0
    n_groups = n_chunks // NBUF
    mesh = plsc.VectorSubcoreMesh(core_axis_name="c", subcore_axis_name="s")

    @functools.partial(
        pl.kernel,
        mesh=mesh,
        out_type=jax.ShapeDtypeStruct((n_samples,), jnp.float32),
        scratch_types=[
            pltpu.VMEM((per_tile,), jnp.float32),
            pltpu.VMEM((per_tile,), jnp.float32),
            pltpu.VMEM((per_tile,), jnp.float32),
            pltpu.VMEM((NBUF, 4, CHUNK), jnp.int32),
            pltpu.VMEM((NBUF, 4, CHUNK), jnp.float32),
            pltpu.VMEM((NBUF, 4, CHUNK), jnp.float32),
        ] + [pltpu.SemaphoreType.DMA] * NBUF,
    )
    def tex_kernel(tex_hbm, xs_hbm, ys_hbm, out_hbm,
                   xs_v, ys_v, out_v, idx_v, val_v, wt_v, *sems):
        wid = lax.axis_index("s") * 2 + lax.axis_index("c")
        base = wid * per_tile
        pltpu.sync_copy(xs_hbm.at[pl.ds(base, per_tile)], xs_v)
        pltpu.sync_copy(ys_hbm.at[pl.ds(base, per_tile)], ys_v)

        def compute_and_fire(ci, slot):
            off = ci * CHUNK
            for i in range(CHUNK // LANES):
                s = off + i * LANES
                d = pl.ds(i * LANES, LANES)
                xf = xs_v[pl.ds(s, LANES)]
                yf = ys_v[pl.ds(s, LANES)]
                gx = ((xf * 2.0 - 1.0) + 1.0) * 0.5 * (W - 1)
                gy = ((yf * 2.0 - 1.0) + 1.0) * 0.5 * (H - 1)
                x0 = gx.astype(jnp.int32)
                y0 = gy.astype(jnp.int32)
                wx1 = gx - x0.astype(jnp.float32)
                wy1 = gy - y0.astype(jnp.float32)
                wx0 = 1.0 - wx1
                wy0 = 1.0 - wy1
                flat = y0 * W + x0
                idx_v[slot, 0, d] = flat
                idx_v[slot, 1, d] = flat + 1
                idx_v[slot, 2, d] = flat + W
                idx_v[slot, 3, d] = flat + (W + 1)
                wt_v[slot, 0, d] = wy0 * wx0
                wt_v[slot, 1, d] = wy0 * wx1
                wt_v[slot, 2, d] = wy1 * wx0
                wt_v[slot, 3, d] = wy1 * wx1
            for c in range(4):
                pltpu.async_copy(tex_hbm.at[idx_v.at[slot, c]],
                                 val_v.at[slot, c], sems[slot])

        def drain_and_combine(ci, slot):
            for c in range(4):
                pltpu.make_async_copy(tex_hbm.at[idx_v.at[slot, c]],
                                      val_v.at[slot, c], sems[slot]).wait()
            off = ci * CHUNK
            for i in range(CHUNK // LANES):
                d = pl.ds(i * LANES, LANES)
                out_v[pl.ds(off + i * LANES, LANES)] = (
                    val_v[slot, 0, d] * wt_v[slot, 0, d]
                    + val_v[slot, 1, d] * wt_v[slot, 1, d]
                    + val_v[slot, 2, d] * wt_v[slot, 2, d]
                    + val_v[slot, 3, d] * wt_v[slot, 3, d]
                )

        for b in range(NBUF - 1):
            compute_and_fire(b, b)

        def loop_body(j, carry):
            cb = j * NBUF
            for b in range(NBUF):
                compute_and_fire(cb + b + (NBUF - 1), (b + NBUF - 1) % NBUF)
                drain_and_combine(cb + b, b)
            return carry

        lax.fori_loop(0, n_groups - 1, loop_body, 0)
        cb = (n_groups - 1) * NBUF
        compute_and_fire(n_chunks - 1, (NBUF - 1) % NBUF)
        for b in range(NBUF):
            drain_and_combine(cb + b, b)

        pltpu.sync_copy(out_v, out_hbm.at[pl.ds(base, per_tile)])

    return tex_kernel


def kernel(x, layer1):
    n, ho, wo = x.shape[0], x.shape[1], x.shape[2]
    n_samples = n * ho * wo
    xs = x[..., 0].reshape(n_samples)
    ys = x[..., 1].reshape(n_samples)
    tex = layer1.reshape(W * H)
    out = _make_kernel(n_samples)(tex, xs, ys)
    return out.reshape(n, 1, ho, wo)
